# baseline (speedup 1.0000x reference)
import functools

import jax
import jax.numpy as jnp
from jax import lax
from jax.experimental import pallas as pl
from jax.experimental.pallas import tpu as pltpu
from jax.experimental.pallas import tpu_sc as plsc

_NUM_NETS = 8
_NUM_PROXIES = 100000
_EMBED_DIM = 16
_B = 26
_LANES = 128
_ROWS = _NUM_PROXIES * _EMBED_DIM // _LANES


def _z_body(scond_ref, perm_ref, t_ref, z_ref):
    z_ref[...] = t_ref[...]


def _z_copy(t3, scond, perm):
    grid_spec = pltpu.PrefetchScalarGridSpec(
        num_scalar_prefetch=2,
        grid=(_B,),
        in_specs=[
            pl.BlockSpec(
                (None, _ROWS, _LANES),
                lambda i, sc, pm: (sc[i], 0, 0),
            ),
        ],
        out_specs=[
            pl.BlockSpec(
                (None, _ROWS, _LANES),
                lambda i, sc, pm: (pm[i], 0, 0),
            ),
        ],
    )
    return pl.pallas_call(
        _z_body,
        grid_spec=grid_spec,
        out_shape=[
            jax.ShapeDtypeStruct((_B, _ROWS, _LANES), jnp.float32),
        ],
        compiler_params=pltpu.CompilerParams(
            dimension_semantics=("arbitrary",),
        ),
    )(scond, perm, t3)[0]


def _x_gather(t128, rowidx, sel):
    mesh = plsc.VectorSubcoreMesh(core_axis_name="c", subcore_axis_name="s")
    chunks = _LANES // _EMBED_DIM

    @functools.partial(
        pl.kernel,
        mesh=mesh,
        out_type=jax.ShapeDtypeStruct((_B, _EMBED_DIM), jnp.float32),
        scratch_types=[
            pltpu.VMEM((_B,), jnp.int32),
            pltpu.VMEM((_B, _EMBED_DIM), jnp.int32),
            pltpu.VMEM((_B, _LANES), jnp.float32),
            pltpu.VMEM((_B, _EMBED_DIM), jnp.float32),
            pltpu.SemaphoreType.DMA,
        ],
    )
    def k(t_hbm, ri_hbm, sel_hbm, out_hbm, ri_v, sel_v, rows_v, out_v, sem):
        wid = lax.axis_index("s") * 2 + lax.axis_index("c")

        @pl.when(wid == 0)
        def _():
            pltpu.sync_copy(ri_hbm, ri_v)
            pltpu.sync_copy(sel_hbm, sel_v)
            pltpu.async_copy(t_hbm.at[ri_v], rows_v, sem).wait()
            for b in range(_B):
                s = sel_v[b, :]
                acc = jnp.zeros((_EMBED_DIM,), jnp.float32)
                for c in range(chunks):
                    chunk = rows_v[b, pl.ds(c * _EMBED_DIM, _EMBED_DIM)]
                    acc = jnp.where(s == c, chunk, acc)
                out_v[b, :] = acc
            pltpu.sync_copy(out_v, out_hbm)

    return k(t128, rowidx, sel)


def kernel(tables, cond, adjs):
    perm = jnp.argsort(cond).astype(jnp.int32)
    scond = cond[perm]

    t3 = tables.reshape(_NUM_NETS, _ROWS, _LANES)
    z = _z_copy(t3, scond, perm).reshape(_B, _NUM_PROXIES, _EMBED_DIM)

    t128 = tables.reshape(_NUM_NETS * _NUM_PROXIES * _EMBED_DIM // _LANES, _LANES)
    g = cond * _NUM_PROXIES + adjs
    rowidx = g // (_LANES // _EMBED_DIM)
    sel = jnp.broadcast_to(
        (g % (_LANES // _EMBED_DIM))[:, None], (_B, _EMBED_DIM)
    ).astype(jnp.int32)
    x = _x_gather(t128, rowidx, sel)
    return (x, z)


# speedup vs baseline: 24.4294x; 24.4294x over previous
import jax
import jax.numpy as jnp
from jax import lax
from jax.experimental import pallas as pl
from jax.experimental.pallas import tpu as pltpu

_NUM_NETS = 8
_NUM_PROXIES = 100000
_EMBED_DIM = 16
_B = 26
_WIN = 128


def _body(scond_ref, perm_ref, sadj_ref, t_ref, win_ref, z_ref, x_ref):
    i = pl.program_id(0)
    z_ref[...] = t_ref[...]

    @pl.when(i == 0)
    def _():
        x_ref[...] = jnp.zeros((_EMBED_DIM, _B), jnp.float32)

    a = sadj_ref[i]
    b = perm_ref[i]
    lane = a % _WIN
    colmask = lax.broadcasted_iota(jnp.int32, (_EMBED_DIM, _WIN), 1) == lane
    col = jnp.sum(
        jnp.where(colmask, win_ref[...], 0.0), axis=1, keepdims=True
    )
    bmask = lax.broadcasted_iota(jnp.int32, (_EMBED_DIM, _B), 1) == b
    x_ref[...] = x_ref[...] + jnp.where(bmask, col, 0.0)


def kernel(tables, cond, adjs):
    perm = jnp.argsort(cond).astype(jnp.int32)
    scond = cond[perm]
    sadj = adjs[perm]

    tt = jnp.transpose(tables, (0, 2, 1))

    grid_spec = pltpu.PrefetchScalarGridSpec(
        num_scalar_prefetch=3,
        grid=(_B,),
        in_specs=[
            pl.BlockSpec(
                (None, _EMBED_DIM, _NUM_PROXIES),
                lambda i, sc, pm, sa: (sc[i], 0, 0),
            ),
            pl.BlockSpec(
                (None, _EMBED_DIM, _WIN),
                lambda i, sc, pm, sa: (sc[i], 0, sa[i] // _WIN),
            ),
        ],
        out_specs=[
            pl.BlockSpec(
                (None, _EMBED_DIM, _NUM_PROXIES),
                lambda i, sc, pm, sa: (pm[i], 0, 0),
            ),
            pl.BlockSpec((_EMBED_DIM, _B), lambda i, sc, pm, sa: (0, 0)),
        ],
    )

    zt, xt = pl.pallas_call(
        _body,
        grid_spec=grid_spec,
        out_shape=[
            jax.ShapeDtypeStruct((_B, _EMBED_DIM, _NUM_PROXIES), jnp.float32),
            jax.ShapeDtypeStruct((_EMBED_DIM, _B), jnp.float32),
        ],
        compiler_params=pltpu.CompilerParams(
            dimension_semantics=("arbitrary",),
        ),
    )(scond, perm, sadj, tt, tt)

    z = jnp.transpose(zt, (0, 2, 1))
    x = xt.T
    return (x, z)
